# Initial kernel scaffold; baseline (speedup 1.0000x reference)
import jax
import jax.numpy as jnp
from jax.experimental import pallas as pl


def kernel(in_tensor, hash_table):
    raise NotImplementedError("write your pallas kernel here")


# speedup vs baseline: 1.4215x; 1.4215x over previous
import numpy as np
import jax
import jax.numpy as jnp
from jax import lax
from jax.experimental import pallas as pl
from jax.experimental.pallas import tpu as pltpu
from jax.experimental.pallas import tpu_sc as plsc

_NUM_LEVELS = 16
_MIN_RES = 16
_MAX_RES = 1024
_TABLE_SIZE = 1 << 19
_MASK = np.int32(_TABLE_SIZE - 1)
_N = 262144
_GROWTH = np.exp((np.log(_MAX_RES) - np.log(_MIN_RES)) / (_NUM_LEVELS - 1))
_SCALES = np.floor(_MIN_RES * _GROWTH ** np.arange(_NUM_LEVELS)).astype(np.float32)
_P2 = np.int32(np.uint32(2654435761))
_P3 = np.int32(805459861)

_NC, _NS = 2, 16
_NW = _NC * _NS
_PPW = _N // _NW
_C = 256
_GPC = _C // 16
_NCHUNK = _PPW // _C
_NR = 8 * _C
_WROWS = _TABLE_SIZE * _NUM_LEVELS // 8


def _loop32(n, body):
    def step(c, _):
        body(c)
        return c + np.int32(1), None

    lax.scan(step, jnp.int32(0), None, length=n)


def _coords_at(coords, p, scale):
    px = coords[0, pl.ds(p, 16)] * scale
    py = coords[1, pl.ds(p, 16)] * scale
    pz = coords[2, pl.ds(p, 16)] * scale
    return px, py, pz


def _compute_idx(level, cbase, coords, ibuf, cbuf):
    scale = jnp.float32(_SCALES[level])
    woff = np.int32(level * (_TABLE_SIZE // 8))

    def g_body(g):
        p = cbase + g * np.int32(16)
        sx, sy, sz = _coords_at(coords, p, scale)
        xi = sx.astype(jnp.int32)
        yi = sy.astype(jnp.int32)
        zi = sz.astype(jnp.int32)
        xc = xi + np.int32(1)
        ypf = yi * _P2
        ypc = ypf + _P2
        zpf = zi * _P3
        zpc = zpf + _P3
        a_cc = xc ^ ypc
        a_cf = xc ^ ypf
        a_fc = xi ^ ypc
        a_ff = xi ^ ypf
        hs = (a_cc ^ zpc, a_cf ^ zpc, a_ff ^ zpc, a_fc ^ zpc,
              a_cc ^ zpf, a_cf ^ zpf, a_ff ^ zpf, a_fc ^ zpf)
        base = g * np.int32(128)
        for k in range(8):
            h = hs[k] & _MASK
            sl = pl.ds(base + np.int32(k * 16), 16)
            ibuf[sl] = (h >> np.int32(3)) + woff
            cbuf[sl] = (h & np.int32(7)) << np.int32(1)

    _loop32(_GPC, g_body)


def _lerp_pass(level, cbase, coords, cbuf, rbuf, obuf):
    scale = jnp.float32(_SCALES[level])
    iota = lax.iota(jnp.int32, 16)

    def g_body(g):
        p = cbase + g * np.int32(16)
        sx, sy, sz = _coords_at(coords, p, scale)
        ox = sx - sx.astype(jnp.int32).astype(jnp.float32)
        oy = sy - sy.astype(jnp.int32).astype(jnp.float32)
        oz = sz - sz.astype(jnp.int32).astype(jnp.float32)
        mx = np.float32(1.0) - ox
        my = np.float32(1.0) - oy
        mz = np.float32(1.0) - oz
        base = g * np.int32(128)
        ridx = g * np.int32(16) + iota
        fv = []
        for k in range(8):
            rk = base + np.int32(k * 16) + iota
            ck = cbuf[pl.ds(base + np.int32(k * 16), 16)]
            fv.append((plsc.load_gather(rbuf, [rk, ck]),
                       plsc.load_gather(rbuf, [rk, ck + np.int32(1)])))
        for f in range(2):
            f03 = fv[0][f] * ox + fv[3][f] * mx
            f12 = fv[1][f] * ox + fv[2][f] * mx
            f56 = fv[5][f] * ox + fv[6][f] * mx
            f47 = fv[4][f] * ox + fv[7][f] * mx
            t = f03 * oy + f12 * my
            b = f47 * oy + f56 * my
            enc = t * oz + b * mz
            ocol = jnp.full((16,), 2 * level + f, jnp.int32)
            plsc.store_scatter(obuf, [ridx, ocol], enc)

    _loop32(_GPC, g_body)


def _hash_encode_body(in_hbm, tab_hbm, out_hbm,
                      coords, idx0, idx1, col0, col1, rows0, rows1, obuf,
                      sem0, sem1, sem_out):
    wid = lax.axis_index("s") * np.int32(_NC) + lax.axis_index("c")
    wbase = wid * np.int32(_PPW)
    pltpu.async_copy(in_hbm.at[:, pl.ds(wbase, _PPW)], coords, sem_out).wait()
    idx_bufs = (idx0, idx1)
    col_bufs = (col0, col1)
    row_bufs = (rows0, rows1)
    sems = (sem0, sem1)

    def chunk_body(c):
        cbase = c * np.int32(_C)
        pbase = wbase + cbase
        _compute_idx(0, cbase, coords, idx_bufs[0], col_bufs[0])
        cur = pltpu.async_copy(tab_hbm.at[idx_bufs[0]], row_bufs[0], sems[0])
        for l in range(_NUM_LEVELS):
            nxt = None
            if l + 1 < _NUM_LEVELS:
                nb = (l + 1) % 2
                _compute_idx(l + 1, cbase, coords, idx_bufs[nb], col_bufs[nb])
                nxt = pltpu.async_copy(tab_hbm.at[idx_bufs[nb]],
                                       row_bufs[nb], sems[nb])
            cur.wait()
            _lerp_pass(l, cbase, coords, col_bufs[l % 2], row_bufs[l % 2], obuf)
            cur = nxt
        pltpu.async_copy(obuf, out_hbm.at[pl.ds(pbase, _C), :], sem_out).wait()

    _loop32(_NCHUNK, chunk_body)


_SCRATCH_TYPES = [
    pltpu.VMEM((3, _PPW), jnp.float32),
    pltpu.VMEM((_NR,), jnp.int32),
    pltpu.VMEM((_NR,), jnp.int32),
    pltpu.VMEM((_NR,), jnp.int32),
    pltpu.VMEM((_NR,), jnp.int32),
    pltpu.VMEM((_NR, 16), jnp.float32),
    pltpu.VMEM((_NR, 16), jnp.float32),
    pltpu.VMEM((_C, 2 * _NUM_LEVELS), jnp.float32),
    pltpu.SemaphoreType.DMA,
    pltpu.SemaphoreType.DMA,
    pltpu.SemaphoreType.DMA,
]

_hash_encode = pl.kernel(
    _hash_encode_body,
    out_type=jax.ShapeDtypeStruct((_N, 2 * _NUM_LEVELS), jnp.float32),
    mesh=plsc.VectorSubcoreMesh(core_axis_name="c", subcore_axis_name="s",
                                num_cores=_NC, num_subcores=_NS),
    scratch_types=_SCRATCH_TYPES,
    compiler_params=pltpu.CompilerParams(needs_layout_passes=False,
                                         use_tc_tiling_on_sc=False),
)


def kernel(in_tensor, hash_table):
    in_t = in_tensor.T.reshape(3, _N)
    tab_wide = hash_table.reshape(_WROWS, 16)
    return _hash_encode(in_t, tab_wide)


# speedup vs baseline: 12.1622x; 8.5556x over previous
import numpy as np
import jax
import jax.numpy as jnp
from jax import lax
from jax.experimental import pallas as pl
from jax.experimental.pallas import tpu as pltpu
from jax.experimental.pallas import tpu_sc as plsc

_NUM_LEVELS = 16
_MIN_RES = 16
_MAX_RES = 1024
_TABLE_SIZE = 1 << 19
_MASK = np.int32(_TABLE_SIZE - 1)
_N = 262144
_GROWTH = np.exp((np.log(_MAX_RES) - np.log(_MIN_RES)) / (_NUM_LEVELS - 1))
_SCALES = np.floor(_MIN_RES * _GROWTH ** np.arange(_NUM_LEVELS)).astype(np.float32)
_P2 = np.int32(np.uint32(2654435761))
_P3 = np.int32(805459861)

_NC, _NS = 2, 16
_NW = _NC * _NS
_PPW = _N // _NW
_C = 128
_GPC = _C // 16
_NCHUNK = _PPW // _C
_NR = 8 * _C
_WROWS = _TABLE_SIZE * _NUM_LEVELS // 8

_NDL = 3
_DRES = tuple(int(r) for r in _SCALES[:_NDL])
_DR1 = tuple(r + 1 for r in _DRES)
_DBASE = (0, _DR1[0] ** 3, _DR1[0] ** 3 + _DR1[1] ** 3)
_DTOT = _DBASE[2] + _DR1[2] ** 3
_CPS = (_DTOT + 8 * _NS - 1) // (8 * _NS) * 8
_DCELLS = _CPS * _NS


def _dense_consts():
    wrs, cos = [], []
    for l in range(_NDL):
        r1 = _DR1[l]
        xs, ys, zs = np.meshgrid(np.arange(r1, dtype=np.int64),
                                 np.arange(r1, dtype=np.int64),
                                 np.arange(r1, dtype=np.int64), indexing="ij")
        h = (xs ^ (ys * 2654435761) ^ (zs * 805459861)) % _TABLE_SIZE
        gh = (h + l * _TABLE_SIZE).ravel()
        wrs.append(gh >> 3)
        cos.append((gh & 7) * 2)
    wr = np.concatenate(wrs)
    co = np.concatenate(cos)
    pad = _DCELLS - wr.size
    wr = np.concatenate([wr, np.zeros(pad, np.int64)]).astype(np.int32)
    co = np.concatenate([co, np.zeros(pad, np.int64)]).astype(np.int32)
    return wr, co


_DWR_NP, _DCO_NP = _dense_consts()


def _loop32(n, body):
    def step(c, _):
        body(c)
        return c + np.int32(1), None

    lax.scan(step, jnp.int32(0), None, length=n)


def _coords_at(coords, p, scale):
    px = coords[0, pl.ds(p, 16)] * scale
    py = coords[1, pl.ds(p, 16)] * scale
    pz = coords[2, pl.ds(p, 16)] * scale
    return px, py, pz


def _offsets(sx, sy, sz):
    xi = sx.astype(jnp.int32)
    yi = sy.astype(jnp.int32)
    zi = sz.astype(jnp.int32)
    ox = sx - xi.astype(jnp.float32)
    oy = sy - yi.astype(jnp.float32)
    oz = sz - zi.astype(jnp.float32)
    return xi, yi, zi, ox, oy, oz


def _lerp_scatter(fv, ox, oy, oz, level, ridx, obuf):
    for f in range(2):
        f03 = fv[3][f] + (fv[0][f] - fv[3][f]) * ox
        f12 = fv[2][f] + (fv[1][f] - fv[2][f]) * ox
        f56 = fv[6][f] + (fv[5][f] - fv[6][f]) * ox
        f47 = fv[7][f] + (fv[4][f] - fv[7][f]) * ox
        t = f12 + (f03 - f12) * oy
        b = f56 + (f47 - f56) * oy
        enc = b + (t - b) * oz
        ocol = jnp.full((16,), 2 * level + f, jnp.int32)
        plsc.store_scatter(obuf, [ridx, ocol], enc)


def _compute_idx(level, coords, ibuf, cbuf):
    scale = jnp.float32(_SCALES[level])
    woff = np.int32(level * (_TABLE_SIZE // 8))

    def g_body(g):
        p = g * np.int32(16)
        sx, sy, sz = _coords_at(coords, p, scale)
        xi = sx.astype(jnp.int32)
        yi = sy.astype(jnp.int32)
        zi = sz.astype(jnp.int32)
        xc = xi + np.int32(1)
        ypf = yi * _P2
        ypc = ypf + _P2
        zpf = zi * _P3
        zpc = zpf + _P3
        a_cc = xc ^ ypc
        a_cf = xc ^ ypf
        a_fc = xi ^ ypc
        a_ff = xi ^ ypf
        hs = (a_cc ^ zpc, a_cf ^ zpc, a_ff ^ zpc, a_fc ^ zpc,
              a_cc ^ zpf, a_cf ^ zpf, a_ff ^ zpf, a_fc ^ zpf)
        base = g * np.int32(128)
        for k in range(8):
            h = hs[k] & _MASK
            sl = pl.ds(base + np.int32(k * 16), 16)
            ibuf[sl] = (h >> np.int32(3)) + woff
            cbuf[sl] = (h & np.int32(7)) << np.int32(1)

    _loop32(_GPC, g_body)


def _lerp_pass(level, coords, cbuf, rbuf, obuf):
    scale = jnp.float32(_SCALES[level])
    iota = lax.iota(jnp.int32, 16)

    def g_body(g):
        p = g * np.int32(16)
        sx, sy, sz = _coords_at(coords, p, scale)
        _, _, _, ox, oy, oz = _offsets(sx, sy, sz)
        base = g * np.int32(128)
        ridx = g * np.int32(16) + iota
        fv = []
        for k in range(8):
            rk = base + np.int32(k * 16) + iota
            ck = cbuf[pl.ds(base + np.int32(k * 16), 16)]
            fv.append((plsc.load_gather(rbuf, [rk, ck]),
                       plsc.load_gather(rbuf, [rk, ck + np.int32(1)])))
        _lerp_scatter(fv, ox, oy, oz, level, ridx, obuf)

    _loop32(_GPC, g_body)


def _dense_pass(level, coords, dense, obuf):
    scale = jnp.float32(_SCALES[level])
    r1 = _DR1[level]
    iota = lax.iota(jnp.int32, 16)

    def g_body(g):
        p = g * np.int32(16)
        sx, sy, sz = _coords_at(coords, p, scale)
        xi, yi, zi, ox, oy, oz = _offsets(sx, sy, sz)
        u_f = xi * np.int32(r1 * r1) + np.int32(_DBASE[level])
        u_c = u_f + np.int32(r1 * r1)
        v_f = yi * np.int32(r1)
        v_c = v_f + np.int32(r1)
        zc = zi + np.int32(1)
        a_cc = u_c + v_c
        a_cf = u_c + v_f
        a_fc = u_f + v_c
        a_ff = u_f + v_f
        cells = (a_cc + zc, a_cf + zc, a_ff + zc, a_fc + zc,
                 a_cc + zi, a_cf + zi, a_ff + zi, a_fc + zi)
        fv = [(plsc.load_gather(dense, [cells[k] << np.int32(1)]),
               plsc.load_gather(dense, [(cells[k] << np.int32(1)) + np.int32(1)]))
              for k in range(8)]
        ridx = g * np.int32(16) + iota
        _lerp_scatter(fv, ox, oy, oz, level, ridx, obuf)

    _loop32(_GPC, g_body)


def _hash_encode_body(in_hbm, tab_hbm, dwr_hbm, dco_hbm, out_hbm,
                      dense, coords, idx0, idx1, col0, col1,
                      rows0, rows1, obuf, spm,
                      sem0, sem1, sem_out, semc0, semc1):
    cid = lax.axis_index("c")
    sid = lax.axis_index("s")
    wid = sid * np.int32(_NC) + cid
    wbase = wid * np.int32(_PPW)
    iota = lax.iota(jnp.int32, 16)
    idx_bufs = (idx0, idx1)
    col_bufs = (col0, col1)
    row_bufs = (rows0, rows1)
    sems = (sem0, sem1)
    semc = (semc0, semc1)

    my0 = sid * np.int32(_CPS)
    off = 0
    for sz in (1024, 1024, _CPS - 2048):
        o32 = my0 + np.int32(off)
        pltpu.async_copy(dwr_hbm.at[pl.ds(o32, sz)],
                         idx0.at[pl.ds(0, sz)], semc0).wait()
        pltpu.async_copy(dco_hbm.at[pl.ds(o32, sz)],
                         col0.at[pl.ds(0, sz)], semc0).wait()
        pltpu.async_copy(tab_hbm.at[idx0.at[pl.ds(0, sz)]],
                         rows0.at[pl.ds(0, sz)], semc0).wait()

        def e_body(g, o32=o32):
            b16 = g * np.int32(16)
            rk = b16 + iota
            ck = col0[pl.ds(b16, 16)]
            v0 = plsc.load_gather(rows0, [rk, ck])
            v1 = plsc.load_gather(rows0, [rk, ck + np.int32(1)])
            c2 = (o32 + b16 + iota) << np.int32(1)
            plsc.store_scatter(dense, [c2], v0)
            plsc.store_scatter(dense, [c2 + np.int32(1)], v1)

        _loop32(sz // 16, e_body)
        off += sz
    pltpu.async_copy(dense.at[pl.ds(my0 * np.int32(2), _CPS * 2)],
                     spm.at[pl.ds(my0 * np.int32(2), _CPS * 2)], semc0).wait()
    plsc.subcore_barrier()
    pltpu.async_copy(spm, dense, semc0).wait()

    def issue_coords(c, par):
        pltpu.async_copy(in_hbm.at[:, pl.ds(wbase + c * np.int32(_C), _C)],
                         coords.at[np.int32(par)], semc[par])

    def wait_coords(c, par):
        pltpu.make_async_copy(in_hbm.at[:, pl.ds(wbase + c * np.int32(_C), _C)],
                              coords.at[np.int32(par)], semc[par]).wait()

    def do_chunk(c, par):
        pbase = wbase + c * np.int32(_C)
        wait_coords(c, par)
        cref = coords.at[np.int32(par)]
        _compute_idx(3, cref, idx_bufs[0], col_bufs[0])
        cur = pltpu.async_copy(tab_hbm.at[idx_bufs[0]], row_bufs[0], sems[0])
        for dl in range(_NDL):
            _dense_pass(dl, cref, dense, obuf)
        for l in range(3, _NUM_LEVELS):
            pp = (l - 3) % 2
            nxt = None
            if l + 1 < _NUM_LEVELS:
                nb = (l - 2) % 2
                _compute_idx(l + 1, cref, idx_bufs[nb], col_bufs[nb])
                nxt = pltpu.async_copy(tab_hbm.at[idx_bufs[nb]],
                                       row_bufs[nb], sems[nb])
            cur.wait()
            _lerp_pass(l, cref, col_bufs[pp], row_bufs[pp], obuf)
            cur = nxt
        pltpu.async_copy(obuf, out_hbm.at[pl.ds(pbase, _C), :], sem_out).wait()

    issue_coords(np.int32(0), 0)
    issue_coords(np.int32(1), 1)

    def pair_body(c2):
        for par in range(2):
            c = c2 * np.int32(2) + np.int32(par)
            do_chunk(c, par)
            issue_coords(jnp.minimum(c + np.int32(2), np.int32(_NCHUNK - 1)),
                         par)

    _loop32(_NCHUNK // 2, pair_body)
    wait_coords(np.int32(_NCHUNK - 1), 0)
    wait_coords(np.int32(_NCHUNK - 1), 1)


_hash_encode = pl.kernel(
    _hash_encode_body,
    out_type=jax.ShapeDtypeStruct((_N, 2 * _NUM_LEVELS), jnp.float32),
    mesh=plsc.VectorSubcoreMesh(core_axis_name="c", subcore_axis_name="s",
                                num_cores=_NC, num_subcores=_NS),
    scratch_types=[
        pltpu.VMEM((_DCELLS * 2,), jnp.float32),
        pltpu.VMEM((2, 3, _C), jnp.float32),
        pltpu.VMEM((_NR,), jnp.int32),
        pltpu.VMEM((_NR,), jnp.int32),
        pltpu.VMEM((_NR,), jnp.int32),
        pltpu.VMEM((_NR,), jnp.int32),
        pltpu.VMEM((_NR, 16), jnp.float32),
        pltpu.VMEM((_NR, 16), jnp.float32),
        pltpu.VMEM((_C, 2 * _NUM_LEVELS), jnp.float32),
        pltpu.VMEM_SHARED((_DCELLS * 2,), jnp.float32),
        pltpu.SemaphoreType.DMA,
        pltpu.SemaphoreType.DMA,
        pltpu.SemaphoreType.DMA,
        pltpu.SemaphoreType.DMA,
        pltpu.SemaphoreType.DMA,
    ],
    compiler_params=pltpu.CompilerParams(needs_layout_passes=False,
                                         use_tc_tiling_on_sc=False),
)


_NB = 16
_BPW = 65536 // _NW


def _relayout_body(nat_hbm, lin_hbm, buf0, buf1, ob0, ob1,
                   si0, si1, so0, so1):
    wid = lax.axis_index("s") * np.int32(_NC) + lax.axis_index("c")
    base = wid * np.int32(_BPW)
    iota = lax.iota(jnp.int32, 16)
    colv = (iota * np.int32(2)) & np.int32(15)
    rowadd = iota >> np.int32(3)
    bufs = (buf0, buf1)
    obufs = (ob0, ob1)
    sin = (si0, si1)
    sout = (so0, so1)
    nchunk = _BPW // _NB

    def issue_in(c, par):
        b0 = base + c * np.int32(_NB)
        pltpu.async_copy(nat_hbm.at[pl.ds(b0, _NB)], bufs[par], sin[par])

    def wait_in(c, par):
        b0 = base + c * np.int32(_NB)
        pltpu.make_async_copy(nat_hbm.at[pl.ds(b0, _NB)], bufs[par],
                              sin[par]).wait()

    def issue_out(c, par):
        b0 = base + c * np.int32(_NB)
        pltpu.async_copy(obufs[par], lin_hbm.at[pl.ds(b0 * np.int32(16),
                                                      _NB * 16), :], sout[par])

    def wait_out(c, par):
        b0 = base + c * np.int32(_NB)
        pltpu.make_async_copy(obufs[par], lin_hbm.at[pl.ds(b0 * np.int32(16),
                                                           _NB * 16), :],
                              sout[par]).wait()

    def transform(par):
        buf, obuf = bufs[par], obufs[par]
        for blk in range(_NB):
            for t in range(8):
                a = buf[blk, 0, pl.ds(t * 16, 16)]
                b = buf[blk, 1, pl.ds(t * 16, 16)]
                rv = rowadd + np.int32(blk * 16 + 2 * t)
                plsc.store_scatter(obuf, [rv, colv], a)
                plsc.store_scatter(obuf, [rv, colv + np.int32(1)], b)

    issue_in(np.int32(0), 0)
    issue_in(np.int32(1), 1)
    wait_in(np.int32(0), 0)
    transform(0)
    issue_out(np.int32(0), 0)
    issue_in(np.int32(2), 0)
    wait_in(np.int32(1), 1)
    transform(1)
    issue_out(np.int32(1), 1)
    issue_in(np.int32(3), 1)

    def pair_body(c2):
        for par in range(2):
            c = c2 * np.int32(2) + np.int32(par)
            wait_in(c, par)
            wait_out(c - np.int32(2), par)
            transform(par)
            issue_out(c, par)
            issue_in(c + np.int32(2), par)

    _loop32(nchunk // 2 - 2, lambda c2: pair_body(c2 + np.int32(1)))

    for par, c in ((0, nchunk - 2), (1, nchunk - 1)):
        ci = np.int32(c)
        wait_in(ci, par)
        wait_out(ci - np.int32(2), par)
        transform(par)
        issue_out(ci, par)
    wait_out(np.int32(nchunk - 2), 0)
    wait_out(np.int32(nchunk - 1), 1)


_relayout = pl.kernel(
    _relayout_body,
    out_type=jax.ShapeDtypeStruct((_WROWS, 16), jnp.float32),
    mesh=plsc.VectorSubcoreMesh(core_axis_name="c", subcore_axis_name="s",
                                num_cores=_NC, num_subcores=_NS),
    scratch_types=[
        pltpu.VMEM((_NB, 2, 128), jnp.float32),
        pltpu.VMEM((_NB, 2, 128), jnp.float32),
        pltpu.VMEM((_NB * 16, 16), jnp.float32),
        pltpu.VMEM((_NB * 16, 16), jnp.float32),
        pltpu.SemaphoreType.DMA,
        pltpu.SemaphoreType.DMA,
        pltpu.SemaphoreType.DMA,
        pltpu.SemaphoreType.DMA,
    ],
    compiler_params=pltpu.CompilerParams(needs_layout_passes=False,
                                         use_tc_tiling_on_sc=False),
)


def kernel(in_tensor, hash_table):
    in_t = in_tensor.T.reshape(3, _N)
    tab_native = hash_table.reshape(65536, 128, 2).transpose(0, 2, 1)
    tab_wide = _relayout(tab_native)
    dwr = jnp.asarray(_DWR_NP)
    dco = jnp.asarray(_DCO_NP)
    return _hash_encode(in_t, tab_wide, dwr, dco)


# speedup vs baseline: 12.2911x; 1.0106x over previous
import numpy as np
import jax
import jax.numpy as jnp
from jax import lax
from jax.experimental import pallas as pl
from jax.experimental.pallas import tpu as pltpu
from jax.experimental.pallas import tpu_sc as plsc

_NUM_LEVELS = 16
_MIN_RES = 16
_MAX_RES = 1024
_TABLE_SIZE = 1 << 19
_MASK = np.int32(_TABLE_SIZE - 1)
_N = 262144
_GROWTH = np.exp((np.log(_MAX_RES) - np.log(_MIN_RES)) / (_NUM_LEVELS - 1))
_SCALES = np.floor(_MIN_RES * _GROWTH ** np.arange(_NUM_LEVELS)).astype(np.float32)
_P2 = np.int32(np.uint32(2654435761))
_P3 = np.int32(805459861)

_NC, _NS = 2, 16
_NW = _NC * _NS
_PPW = _N // _NW
_C = 128
_GPC = _C // 16
_NCHUNK = _PPW // _C
_NR = 8 * _C
_WROWS = _TABLE_SIZE * _NUM_LEVELS // 8

_NDL = 3
_DRES = tuple(int(r) for r in _SCALES[:_NDL])
_DR1 = tuple(r + 1 for r in _DRES)
_DBASE = (0, _DR1[0] ** 3, _DR1[0] ** 3 + _DR1[1] ** 3)
_DTOT = _DBASE[2] + _DR1[2] ** 3
_CPS = (_DTOT + 8 * _NS - 1) // (8 * _NS) * 8
_DCELLS = _CPS * _NS


def _dense_consts():
    wrs, cos = [], []
    for l in range(_NDL):
        r1 = _DR1[l]
        xs, ys, zs = np.meshgrid(np.arange(r1, dtype=np.int64),
                                 np.arange(r1, dtype=np.int64),
                                 np.arange(r1, dtype=np.int64), indexing="ij")
        h = (xs ^ (ys * 2654435761) ^ (zs * 805459861)) % _TABLE_SIZE
        gh = (h + l * _TABLE_SIZE).ravel()
        wrs.append(gh >> 3)
        cos.append((gh & 7) * 2)
    wr = np.concatenate(wrs)
    co = np.concatenate(cos)
    pad = _DCELLS - wr.size
    wr = np.concatenate([wr, np.zeros(pad, np.int64)]).astype(np.int32)
    co = np.concatenate([co, np.zeros(pad, np.int64)]).astype(np.int32)
    return wr, co


_DWR_NP, _DCO_NP = _dense_consts()


def _loop32(n, body):
    def step(c, _):
        body(c)
        return c + np.int32(1), None

    lax.scan(step, jnp.int32(0), None, length=n)


def _coords_at(coords, p, scale):
    px = coords[0, pl.ds(p, 16)] * scale
    py = coords[1, pl.ds(p, 16)] * scale
    pz = coords[2, pl.ds(p, 16)] * scale
    return px, py, pz


def _offsets(sx, sy, sz):
    xi = sx.astype(jnp.int32)
    yi = sy.astype(jnp.int32)
    zi = sz.astype(jnp.int32)
    ox = sx - xi.astype(jnp.float32)
    oy = sy - yi.astype(jnp.float32)
    oz = sz - zi.astype(jnp.float32)
    return xi, yi, zi, ox, oy, oz


def _lerp_scatter(fv, ox, oy, oz, level, ridx, obuf):
    for f in range(2):
        f03 = fv[3][f] + (fv[0][f] - fv[3][f]) * ox
        f12 = fv[2][f] + (fv[1][f] - fv[2][f]) * ox
        f56 = fv[6][f] + (fv[5][f] - fv[6][f]) * ox
        f47 = fv[7][f] + (fv[4][f] - fv[7][f]) * ox
        t = f12 + (f03 - f12) * oy
        b = f56 + (f47 - f56) * oy
        enc = b + (t - b) * oz
        ocol = jnp.full((16,), 2 * level + f, jnp.int32)
        plsc.store_scatter(obuf, [ridx, ocol], enc)


def _compute_idx(level, coords, ibuf, cbuf):
    scale = jnp.float32(_SCALES[level])
    woff = np.int32(level * (_TABLE_SIZE // 8))

    def g_body(g):
        p = g * np.int32(16)
        sx, sy, sz = _coords_at(coords, p, scale)
        xi = sx.astype(jnp.int32)
        yi = sy.astype(jnp.int32)
        zi = sz.astype(jnp.int32)
        xc = xi + np.int32(1)
        ypf = yi * _P2
        ypc = ypf + _P2
        zpf = zi * _P3
        zpc = zpf + _P3
        a_cc = xc ^ ypc
        a_cf = xc ^ ypf
        a_fc = xi ^ ypc
        a_ff = xi ^ ypf
        hs = (a_cc ^ zpc, a_cf ^ zpc, a_ff ^ zpc, a_fc ^ zpc,
              a_cc ^ zpf, a_cf ^ zpf, a_ff ^ zpf, a_fc ^ zpf)
        base = g * np.int32(128)
        for k in range(8):
            h = hs[k] & _MASK
            sl = pl.ds(base + np.int32(k * 16), 16)
            ibuf[sl] = (h >> np.int32(3)) + woff
            cbuf[sl] = (h & np.int32(7)) << np.int32(1)

    _loop32(_GPC, g_body)


def _lerp_pass(level, coords, cbuf, rbuf, obuf):
    scale = jnp.float32(_SCALES[level])
    iota = lax.iota(jnp.int32, 16)

    def g_body(g):
        p = g * np.int32(16)
        sx, sy, sz = _coords_at(coords, p, scale)
        _, _, _, ox, oy, oz = _offsets(sx, sy, sz)
        base = g * np.int32(128)
        ridx = g * np.int32(16) + iota
        fv = []
        for k in range(8):
            rk = base + np.int32(k * 16) + iota
            ck = cbuf[pl.ds(base + np.int32(k * 16), 16)]
            fv.append((plsc.load_gather(rbuf, [rk, ck]),
                       plsc.load_gather(rbuf, [rk, ck + np.int32(1)])))
        _lerp_scatter(fv, ox, oy, oz, level, ridx, obuf)

    _loop32(_GPC, g_body)


def _dense_pass(level, coords, dense, obuf):
    scale = jnp.float32(_SCALES[level])
    r1 = _DR1[level]
    iota = lax.iota(jnp.int32, 16)

    def g_body(g):
        p = g * np.int32(16)
        sx, sy, sz = _coords_at(coords, p, scale)
        xi, yi, zi, ox, oy, oz = _offsets(sx, sy, sz)
        u_f = xi * np.int32(r1 * r1) + np.int32(_DBASE[level])
        u_c = u_f + np.int32(r1 * r1)
        v_f = yi * np.int32(r1)
        v_c = v_f + np.int32(r1)
        zc = zi + np.int32(1)
        a_cc = u_c + v_c
        a_cf = u_c + v_f
        a_fc = u_f + v_c
        a_ff = u_f + v_f
        cells = (a_cc + zc, a_cf + zc, a_ff + zc, a_fc + zc,
                 a_cc + zi, a_cf + zi, a_ff + zi, a_fc + zi)
        fv = [(plsc.load_gather(dense, [cells[k] << np.int32(1)]),
               plsc.load_gather(dense, [(cells[k] << np.int32(1)) + np.int32(1)]))
              for k in range(8)]
        ridx = g * np.int32(16) + iota
        _lerp_scatter(fv, ox, oy, oz, level, ridx, obuf)

    _loop32(_GPC, g_body)


def _hash_encode_body(in_hbm, tab_hbm, dwr_hbm, dco_hbm, out_hbm,
                      dense, coords, idx0, idx1, col0, col1,
                      rows0, rows1, obuf, spm,
                      sem0, sem1, sem_out, semc0, semc1):
    cid = lax.axis_index("c")
    sid = lax.axis_index("s")
    wid = sid * np.int32(_NC) + cid
    wbase = wid * np.int32(_PPW)
    iota = lax.iota(jnp.int32, 16)
    idx_bufs = (idx0, idx1)
    col_bufs = (col0, col1)
    row_bufs = (rows0, rows1)
    sems = (sem0, sem1)
    semc = (semc0, semc1)

    my0 = sid * np.int32(_CPS)
    off = 0
    for sz in (1024, 1024, _CPS - 2048):
        o32 = my0 + np.int32(off)
        pltpu.async_copy(dwr_hbm.at[pl.ds(o32, sz)],
                         idx0.at[pl.ds(0, sz)], semc0).wait()
        pltpu.async_copy(dco_hbm.at[pl.ds(o32, sz)],
                         col0.at[pl.ds(0, sz)], semc0).wait()
        pltpu.async_copy(tab_hbm.at[idx0.at[pl.ds(0, sz)]],
                         rows0.at[pl.ds(0, sz)], semc0).wait()

        def e_body(g, o32=o32):
            b16 = g * np.int32(16)
            rk = b16 + iota
            ck = col0[pl.ds(b16, 16)]
            v0 = plsc.load_gather(rows0, [rk, ck])
            v1 = plsc.load_gather(rows0, [rk, ck + np.int32(1)])
            c2 = (o32 + b16 + iota) << np.int32(1)
            plsc.store_scatter(dense, [c2], v0)
            plsc.store_scatter(dense, [c2 + np.int32(1)], v1)

        _loop32(sz // 16, e_body)
        off += sz
    pltpu.async_copy(dense.at[pl.ds(my0 * np.int32(2), _CPS * 2)],
                     spm.at[pl.ds(my0 * np.int32(2), _CPS * 2)], semc0).wait()
    plsc.subcore_barrier()
    pltpu.async_copy(spm, dense, semc0).wait()

    def issue_coords(c, par):
        pltpu.async_copy(in_hbm.at[:, pl.ds(wbase + c * np.int32(_C), _C)],
                         coords.at[np.int32(par)], semc[par])

    def wait_coords(c, par):
        pltpu.make_async_copy(in_hbm.at[:, pl.ds(wbase + c * np.int32(_C), _C)],
                              coords.at[np.int32(par)], semc[par]).wait()

    def start_gather(b):
        return pltpu.async_copy(tab_hbm.at[idx_bufs[b]], row_bufs[b], sems[b])

    def do_chunk(c, par):
        pbase = wbase + c * np.int32(_C)
        cref = coords.at[np.int32(par)]
        cur = None
        for dl in range(_NDL):
            _dense_pass(dl, cref, dense, obuf)
        for l in range(3, _NUM_LEVELS):
            pp = (l - 3 + par) % 2
            nxt = None
            if l + 1 < _NUM_LEVELS:
                nb = (l - 2 + par) % 2
                _compute_idx(l + 1, cref, idx_bufs[nb], col_bufs[nb])
                nxt = start_gather(nb)
            else:
                npar = 1 - par

                @pl.when(c + np.int32(1) < np.int32(_NCHUNK))
                def _():
                    wait_coords(c + np.int32(1), npar)
                    ncref = coords.at[np.int32(npar)]
                    _compute_idx(3, ncref, idx_bufs[npar], col_bufs[npar])
                    start_gather(npar)
            if cur is None:
                pltpu.make_async_copy(tab_hbm.at[idx_bufs[pp]],
                                      row_bufs[pp], sems[pp]).wait()
            else:
                cur.wait()
            _lerp_pass(l, cref, col_bufs[pp], row_bufs[pp], obuf)
            cur = nxt
        pltpu.async_copy(obuf, out_hbm.at[pl.ds(pbase, _C), :], sem_out).wait()
        issue_coords(jnp.minimum(c + np.int32(2), np.int32(_NCHUNK - 1)), par)

    issue_coords(np.int32(0), 0)
    issue_coords(np.int32(1), 1)
    wait_coords(np.int32(0), 0)
    _compute_idx(3, coords.at[np.int32(0)], idx_bufs[0], col_bufs[0])
    start_gather(0)

    def pair_body(c2):
        for par in range(2):
            do_chunk(c2 * np.int32(2) + np.int32(par), par)

    _loop32(_NCHUNK // 2, pair_body)
    wait_coords(np.int32(_NCHUNK - 1), 0)
    wait_coords(np.int32(_NCHUNK - 1), 1)


_hash_encode = pl.kernel(
    _hash_encode_body,
    out_type=jax.ShapeDtypeStruct((_N, 2 * _NUM_LEVELS), jnp.float32),
    mesh=plsc.VectorSubcoreMesh(core_axis_name="c", subcore_axis_name="s",
                                num_cores=_NC, num_subcores=_NS),
    scratch_types=[
        pltpu.VMEM((_DCELLS * 2,), jnp.float32),
        pltpu.VMEM((2, 3, _C), jnp.float32),
        pltpu.VMEM((_NR,), jnp.int32),
        pltpu.VMEM((_NR,), jnp.int32),
        pltpu.VMEM((_NR,), jnp.int32),
        pltpu.VMEM((_NR,), jnp.int32),
        pltpu.VMEM((_NR, 16), jnp.float32),
        pltpu.VMEM((_NR, 16), jnp.float32),
        pltpu.VMEM((_C, 2 * _NUM_LEVELS), jnp.float32),
        pltpu.VMEM_SHARED((_DCELLS * 2,), jnp.float32),
        pltpu.SemaphoreType.DMA,
        pltpu.SemaphoreType.DMA,
        pltpu.SemaphoreType.DMA,
        pltpu.SemaphoreType.DMA,
        pltpu.SemaphoreType.DMA,
    ],
    compiler_params=pltpu.CompilerParams(needs_layout_passes=False,
                                         use_tc_tiling_on_sc=False),
)


_NB = 16
_BPW = 65536 // _NW


def _relayout_body(nat_hbm, lin_hbm, buf0, buf1, ob0, ob1,
                   si0, si1, so0, so1):
    wid = lax.axis_index("s") * np.int32(_NC) + lax.axis_index("c")
    base = wid * np.int32(_BPW)
    iota = lax.iota(jnp.int32, 16)
    colv = (iota * np.int32(2)) & np.int32(15)
    rowadd = iota >> np.int32(3)
    bufs = (buf0, buf1)
    obufs = (ob0, ob1)
    sin = (si0, si1)
    sout = (so0, so1)
    nchunk = _BPW // _NB

    def issue_in(c, par):
        b0 = base + c * np.int32(_NB)
        pltpu.async_copy(nat_hbm.at[pl.ds(b0, _NB)], bufs[par], sin[par])

    def wait_in(c, par):
        b0 = base + c * np.int32(_NB)
        pltpu.make_async_copy(nat_hbm.at[pl.ds(b0, _NB)], bufs[par],
                              sin[par]).wait()

    def issue_out(c, par):
        b0 = base + c * np.int32(_NB)
        pltpu.async_copy(obufs[par], lin_hbm.at[pl.ds(b0 * np.int32(16),
                                                      _NB * 16), :], sout[par])

    def wait_out(c, par):
        b0 = base + c * np.int32(_NB)
        pltpu.make_async_copy(obufs[par], lin_hbm.at[pl.ds(b0 * np.int32(16),
                                                           _NB * 16), :],
                              sout[par]).wait()

    def transform(par):
        buf, obuf = bufs[par], obufs[par]
        for blk in range(_NB):
            for t in range(8):
                a = buf[blk, 0, pl.ds(t * 16, 16)]
                b = buf[blk, 1, pl.ds(t * 16, 16)]
                rv = rowadd + np.int32(blk * 16 + 2 * t)
                plsc.store_scatter(obuf, [rv, colv], a)
                plsc.store_scatter(obuf, [rv, colv + np.int32(1)], b)

    issue_in(np.int32(0), 0)
    issue_in(np.int32(1), 1)
    wait_in(np.int32(0), 0)
    transform(0)
    issue_out(np.int32(0), 0)
    issue_in(np.int32(2), 0)
    wait_in(np.int32(1), 1)
    transform(1)
    issue_out(np.int32(1), 1)
    issue_in(np.int32(3), 1)

    def pair_body(c2):
        for par in range(2):
            c = c2 * np.int32(2) + np.int32(par)
            wait_in(c, par)
            wait_out(c - np.int32(2), par)
            transform(par)
            issue_out(c, par)
            issue_in(c + np.int32(2), par)

    _loop32(nchunk // 2 - 2, lambda c2: pair_body(c2 + np.int32(1)))

    for par, c in ((0, nchunk - 2), (1, nchunk - 1)):
        ci = np.int32(c)
        wait_in(ci, par)
        wait_out(ci - np.int32(2), par)
        transform(par)
        issue_out(ci, par)
    wait_out(np.int32(nchunk - 2), 0)
    wait_out(np.int32(nchunk - 1), 1)


_relayout = pl.kernel(
    _relayout_body,
    out_type=jax.ShapeDtypeStruct((_WROWS, 16), jnp.float32),
    mesh=plsc.VectorSubcoreMesh(core_axis_name="c", subcore_axis_name="s",
                                num_cores=_NC, num_subcores=_NS),
    scratch_types=[
        pltpu.VMEM((_NB, 2, 128), jnp.float32),
        pltpu.VMEM((_NB, 2, 128), jnp.float32),
        pltpu.VMEM((_NB * 16, 16), jnp.float32),
        pltpu.VMEM((_NB * 16, 16), jnp.float32),
        pltpu.SemaphoreType.DMA,
        pltpu.SemaphoreType.DMA,
        pltpu.SemaphoreType.DMA,
        pltpu.SemaphoreType.DMA,
    ],
    compiler_params=pltpu.CompilerParams(needs_layout_passes=False,
                                         use_tc_tiling_on_sc=False),
)


def kernel(in_tensor, hash_table):
    in_t = in_tensor.T.reshape(3, _N)
    tab_native = hash_table.reshape(65536, 128, 2).transpose(0, 2, 1)
    tab_wide = _relayout(tab_native)
    dwr = jnp.asarray(_DWR_NP)
    dco = jnp.asarray(_DCO_NP)
    return _hash_encode(in_t, tab_wide, dwr, dco)
